# Initial kernel scaffold; baseline (speedup 1.0000x reference)
import functools

import jax
import jax.numpy as jnp
from jax import lax
from jax.experimental import pallas as pl
from jax.experimental.pallas import tpu as pltpu
from jax.experimental.pallas import tpu_sc as plsc

CUTOFF = 8.0
NUM_WORKERS = 32
LANES = 16


def _sc_pair_dist2(pos_t, idx_i, idx_j):
    n_atoms = pos_t.shape[1]
    p_pad = idx_i.shape[0]
    per_w = p_pad // NUM_WORKERS
    chunks = per_w // LANES
    mesh = plsc.VectorSubcoreMesh(core_axis_name="c", subcore_axis_name="s")

    @functools.partial(
        pl.kernel,
        out_type=jax.ShapeDtypeStruct((p_pad,), jnp.float32),
        mesh=mesh,
        scratch_types=[
            pltpu.VMEM((n_atoms,), jnp.float32),
            pltpu.VMEM((per_w,), jnp.int32),
            pltpu.VMEM((per_w,), jnp.int32),
            pltpu.VMEM((per_w,), jnp.float32),
        ],
    )
    def sc_kernel(pos_hbm, ii_hbm, jj_hbm, d2_hbm, tab_v, ii_v, jj_v, d2_v):
        wid = lax.axis_index("s") * 2 + lax.axis_index("c")
        base = wid * per_w
        pltpu.sync_copy(ii_hbm.at[pl.ds(base, per_w)], ii_v)
        pltpu.sync_copy(jj_hbm.at[pl.ds(base, per_w)], jj_v)
        for c in range(3):
            pltpu.sync_copy(pos_hbm.at[c], tab_v)

            def body(k, _, first=(c == 0)):
                off = k * LANES
                ii = ii_v[pl.ds(off, LANES)]
                jj = jj_v[pl.ds(off, LANES)]
                xi = plsc.load_gather(tab_v, [ii])
                xj = plsc.load_gather(tab_v, [jj])
                d = xj - xi
                if first:
                    d2_v[pl.ds(off, LANES)] = d * d
                else:
                    d2_v[pl.ds(off, LANES)] = d2_v[pl.ds(off, LANES)] + d * d
                return 0

            lax.fori_loop(0, chunks, body, 0)
        pltpu.sync_copy(d2_v, d2_hbm.at[pl.ds(base, per_w)])

    return sc_kernel(pos_t, idx_i, idx_j)


def _tc_rbf(d2_col, centers_row, widths_row):
    p, _ = d2_col.shape
    n_rbf = centers_row.shape[1]
    blk = 6400
    grid = p // blk

    def body(d2_ref, c_ref, w_ref, d_ref, cut_ref, rbf_ref):
        d2 = d2_ref[...]
        d = jnp.sqrt(d2)
        x = d * (1.0 / CUTOFF)
        x3 = x * x * x
        f = 1.0 + x3 * (-10.0 + x * (15.0 - 6.0 * x))
        cut_ref[...] = jnp.where(d < CUTOFF, f, jnp.zeros_like(f))
        d_ref[...] = d
        diff = d - c_ref[...]
        rbf_ref[...] = jnp.exp(-w_ref[...] * diff * diff)

    return pl.pallas_call(
        body,
        grid=(grid,),
        in_specs=[
            pl.BlockSpec((blk, 1), lambda i: (i, 0)),
            pl.BlockSpec((1, n_rbf), lambda i: (0, 0)),
            pl.BlockSpec((1, n_rbf), lambda i: (0, 0)),
        ],
        out_specs=[
            pl.BlockSpec((blk, 1), lambda i: (i, 0)),
            pl.BlockSpec((blk, 1), lambda i: (i, 0)),
            pl.BlockSpec((blk, n_rbf), lambda i: (i, 0)),
        ],
        out_shape=[
            jax.ShapeDtypeStruct((p, 1), jnp.float32),
            jax.ShapeDtypeStruct((p, 1), jnp.float32),
            jax.ShapeDtypeStruct((p, n_rbf), jnp.float32),
        ],
    )(d2_col, centers_row, widths_row)


def _tc_features(ids_col, table):
    n, _ = ids_col.shape
    v, d = table.shape
    blk = 2000
    grid = n // blk

    def body(id_ref, t_ref, out_ref):
        ids = id_ref[...]
        iot = lax.broadcasted_iota(jnp.int32, (1, v), 1)
        onehot = (ids == iot).astype(jnp.float32)
        out_ref[...] = jnp.dot(onehot, t_ref[...],
                               preferred_element_type=jnp.float32)

    return pl.pallas_call(
        body,
        grid=(grid,),
        in_specs=[
            pl.BlockSpec((blk, 1), lambda i: (i, 0)),
            pl.BlockSpec((v, d), lambda i: (0, 0)),
        ],
        out_specs=pl.BlockSpec((blk, d), lambda i: (i, 0)),
        out_shape=jax.ShapeDtypeStruct((n, d), jnp.float32),
    )(ids_col, table)


def kernel(atomic_numbers, positions, idx_i, idx_j,
           atom_features, rbf_centers, rbf_widths):
    p = idx_i.shape[0]
    n_rbf = rbf_centers.shape[0]

    step = NUM_WORKERS * LANES
    p_pad = -(-p // step) * step
    pad = p_pad - p
    ii = idx_i.astype(jnp.int32)
    jj = idx_j.astype(jnp.int32)
    if pad:
        zeros = jnp.zeros((pad,), jnp.int32)
        ii = jnp.concatenate([ii, zeros])
        jj = jnp.concatenate([jj, zeros])

    pos_t = positions.astype(jnp.float32).T
    d2 = _sc_pair_dist2(pos_t, ii, jj)[:p]

    d_col, cut_col, rbfs = _tc_rbf(
        d2.reshape(p, 1),
        rbf_centers.astype(jnp.float32).reshape(1, n_rbf),
        rbf_widths.astype(jnp.float32).reshape(1, n_rbf),
    )
    distances = d_col.reshape(p)
    cutoffs = cut_col.reshape(p)

    vocab, dim = atom_features.shape
    v_pad = -(-vocab // 8) * 8
    table = atom_features.astype(jnp.float32)
    if v_pad != vocab:
        table = jnp.concatenate(
            [table, jnp.zeros((v_pad - vocab, dim), jnp.float32)])
    features = _tc_features(
        atomic_numbers.astype(jnp.int32).reshape(-1, 1), table)

    return (features, distances, cutoffs, rbfs, distances)


# speedup vs baseline: 2.9875x; 2.9875x over previous
import functools

import jax
import jax.numpy as jnp
from jax import lax
from jax.experimental import pallas as pl
from jax.experimental.pallas import tpu as pltpu
from jax.experimental.pallas import tpu_sc as plsc

CUTOFF = 8.0
NUM_WORKERS = 32
LANES = 16


def _sc_pair_dist2(xs, ys, zs, idx_i, idx_j):
    n_atoms = xs.shape[0]
    p_pad = idx_i.shape[0]
    per_w = p_pad // NUM_WORKERS
    chunks = per_w // LANES
    mesh = plsc.VectorSubcoreMesh(core_axis_name="c", subcore_axis_name="s")

    @functools.partial(
        pl.kernel,
        out_type=jax.ShapeDtypeStruct((p_pad,), jnp.float32),
        mesh=mesh,
        scratch_types=[
            pltpu.VMEM((n_atoms,), jnp.float32),
            pltpu.VMEM((per_w,), jnp.int32),
            pltpu.VMEM((per_w,), jnp.int32),
            pltpu.VMEM((per_w,), jnp.float32),
        ],
        compiler_params=pltpu.CompilerParams(needs_layout_passes=False),
    )
    def sc_kernel(x_hbm, y_hbm, z_hbm, ii_hbm, jj_hbm, d2_hbm,
                  tab_v, ii_v, jj_v, d2_v):
        wid = lax.axis_index("s") * 2 + lax.axis_index("c")
        base = wid * per_w
        pltpu.sync_copy(ii_hbm.at[pl.ds(base, per_w)], ii_v)
        pltpu.sync_copy(jj_hbm.at[pl.ds(base, per_w)], jj_v)
        for c, coord_hbm in enumerate((x_hbm, y_hbm, z_hbm)):
            pltpu.sync_copy(coord_hbm, tab_v)

            def body(k, _, first=(c == 0)):
                off = k * LANES
                ii = ii_v[pl.ds(off, LANES)]
                jj = jj_v[pl.ds(off, LANES)]
                xi = plsc.load_gather(tab_v, [ii])
                xj = plsc.load_gather(tab_v, [jj])
                d = xj - xi
                if first:
                    d2_v[pl.ds(off, LANES)] = d * d
                else:
                    d2_v[pl.ds(off, LANES)] = d2_v[pl.ds(off, LANES)] + d * d
                return 0

            lax.fori_loop(0, chunks, body, 0)
        pltpu.sync_copy(d2_v, d2_hbm.at[pl.ds(base, per_w)])

    return sc_kernel(xs, ys, zs, idx_i, idx_j)


def _tc_rbf(d2_col, centers_row, widths_row):
    p, _ = d2_col.shape
    n_rbf = centers_row.shape[1]
    blk = 6400
    grid = p // blk

    def body(d2_ref, c_ref, w_ref, d_ref, cut_ref, rbf_ref):
        d2 = d2_ref[...]
        d = jnp.sqrt(d2)
        x = d * (1.0 / CUTOFF)
        x3 = x * x * x
        f = 1.0 + x3 * (-10.0 + x * (15.0 - 6.0 * x))
        cut_ref[...] = jnp.where(d < CUTOFF, f, jnp.zeros_like(f))
        d_ref[...] = d
        diff = d - c_ref[...]
        rbf_ref[...] = jnp.exp(-w_ref[...] * diff * diff)

    return pl.pallas_call(
        body,
        grid=(grid,),
        in_specs=[
            pl.BlockSpec((blk, 1), lambda i: (i, 0)),
            pl.BlockSpec((1, n_rbf), lambda i: (0, 0)),
            pl.BlockSpec((1, n_rbf), lambda i: (0, 0)),
        ],
        out_specs=[
            pl.BlockSpec((blk, 1), lambda i: (i, 0)),
            pl.BlockSpec((blk, 1), lambda i: (i, 0)),
            pl.BlockSpec((blk, n_rbf), lambda i: (i, 0)),
        ],
        out_shape=[
            jax.ShapeDtypeStruct((p, 1), jnp.float32),
            jax.ShapeDtypeStruct((p, 1), jnp.float32),
            jax.ShapeDtypeStruct((p, n_rbf), jnp.float32),
        ],
    )(d2_col, centers_row, widths_row)


def _tc_features(ids_col, table):
    n, _ = ids_col.shape
    v, d = table.shape
    blk = 2000
    grid = n // blk

    def body(id_ref, t_ref, out_ref):
        ids = id_ref[...]
        iot = lax.broadcasted_iota(jnp.int32, (1, v), 1)
        onehot = (ids == iot).astype(jnp.float32)
        out_ref[...] = jnp.dot(onehot, t_ref[...],
                               preferred_element_type=jnp.float32)

    return pl.pallas_call(
        body,
        grid=(grid,),
        in_specs=[
            pl.BlockSpec((blk, 1), lambda i: (i, 0)),
            pl.BlockSpec((v, d), lambda i: (0, 0)),
        ],
        out_specs=pl.BlockSpec((blk, d), lambda i: (i, 0)),
        out_shape=jax.ShapeDtypeStruct((n, d), jnp.float32),
    )(ids_col, table)


def kernel(atomic_numbers, positions, idx_i, idx_j,
           atom_features, rbf_centers, rbf_widths):
    p = idx_i.shape[0]
    n_rbf = rbf_centers.shape[0]

    step = NUM_WORKERS * LANES
    p_pad = -(-p // step) * step
    pad = p_pad - p
    ii = idx_i.astype(jnp.int32)
    jj = idx_j.astype(jnp.int32)
    if pad:
        zeros = jnp.zeros((pad,), jnp.int32)
        ii = jnp.concatenate([ii, zeros])
        jj = jnp.concatenate([jj, zeros])

    pos = positions.astype(jnp.float32)
    d2 = _sc_pair_dist2(pos[:, 0], pos[:, 1], pos[:, 2], ii, jj)[:p]

    d_col, cut_col, rbfs = _tc_rbf(
        d2.reshape(p, 1),
        rbf_centers.astype(jnp.float32).reshape(1, n_rbf),
        rbf_widths.astype(jnp.float32).reshape(1, n_rbf),
    )
    distances = d_col.reshape(p)
    cutoffs = cut_col.reshape(p)

    vocab, dim = atom_features.shape
    v_pad = -(-vocab // 8) * 8
    table = atom_features.astype(jnp.float32)
    if v_pad != vocab:
        table = jnp.concatenate(
            [table, jnp.zeros((v_pad - vocab, dim), jnp.float32)])
    features = _tc_features(
        atomic_numbers.astype(jnp.int32).reshape(-1, 1), table)

    return (features, distances, cutoffs, rbfs, distances)


# speedup vs baseline: 3.1516x; 1.0549x over previous
import functools

import jax
import jax.numpy as jnp
from jax import lax
from jax.experimental import pallas as pl
from jax.experimental.pallas import tpu as pltpu
from jax.experimental.pallas import tpu_sc as plsc

CUTOFF = 8.0
NUM_WORKERS = 32
LANES = 16


def _sc_pair_dist2(xs, ys, zs, idx_i, idx_j):
    n_atoms = xs.shape[0]
    p_pad = idx_i.shape[0]
    per_w = p_pad // NUM_WORKERS
    chunks = per_w // LANES
    mesh = plsc.VectorSubcoreMesh(core_axis_name="c", subcore_axis_name="s")

    @functools.partial(
        pl.kernel,
        out_type=jax.ShapeDtypeStruct((p_pad,), jnp.float32),
        mesh=mesh,
        scratch_types=[
            pltpu.VMEM((n_atoms,), jnp.float32),
            pltpu.VMEM((per_w,), jnp.int32),
            pltpu.VMEM((per_w,), jnp.int32),
            pltpu.VMEM((per_w,), jnp.float32),
        ],
        compiler_params=pltpu.CompilerParams(needs_layout_passes=False),
    )
    def sc_kernel(x_hbm, y_hbm, z_hbm, ii_hbm, jj_hbm, d2_hbm,
                  tab_v, ii_v, jj_v, d2_v):
        wid = lax.axis_index("s") * 2 + lax.axis_index("c")
        base = wid * per_w
        pltpu.sync_copy(ii_hbm.at[pl.ds(base, per_w)], ii_v)
        pltpu.sync_copy(jj_hbm.at[pl.ds(base, per_w)], jj_v)
        for c, coord_hbm in enumerate((x_hbm, y_hbm, z_hbm)):
            pltpu.sync_copy(coord_hbm, tab_v)

            def body(k, _, first=(c == 0)):
                off = k * LANES
                ii = ii_v[pl.ds(off, LANES)]
                jj = jj_v[pl.ds(off, LANES)]
                xi = plsc.load_gather(tab_v, [ii])
                xj = plsc.load_gather(tab_v, [jj])
                d = xj - xi
                if first:
                    d2_v[pl.ds(off, LANES)] = d * d
                else:
                    d2_v[pl.ds(off, LANES)] = d2_v[pl.ds(off, LANES)] + d * d
                return 0

            lax.fori_loop(0, chunks, body, 0)
        pltpu.sync_copy(d2_v, d2_hbm.at[pl.ds(base, per_w)])

    return sc_kernel(xs, ys, zs, idx_i, idx_j)


def _tc_rbf(d2_pairs, centers_row, widths_row):
    half, _ = d2_pairs.shape
    n_rbf = centers_row.shape[1]
    blk = 4000
    grid = half // blk

    def body(d2_ref, c_ref, w_ref, d_ref, cut_ref, rbf_ref):
        d2 = d2_ref[...]
        dd = jnp.sqrt(d2)
        d_ref[...] = dd
        x = dd * (1.0 / CUTOFF)
        x3 = x * x * x
        f = 1.0 + x3 * (-10.0 + x * (15.0 - 6.0 * x))
        cut_ref[...] = jnp.where(dd < CUTOFF, f, jnp.zeros_like(f))
        row = lax.broadcasted_iota(jnp.int32, (2, 2 * n_rbf), 0)
        col = lax.broadcasted_iota(jnp.int32, (2, 2 * n_rbf), 1)
        rep = (row == col // n_rbf).astype(jnp.float32)
        big = jnp.dot(dd, rep, preferred_element_type=jnp.float32,
                      precision=lax.Precision.HIGHEST)
        c2 = jnp.concatenate([c_ref[...], c_ref[...]], axis=1)
        w2 = jnp.concatenate([w_ref[...], w_ref[...]], axis=1)
        z = big - c2
        rbf_ref[...] = jnp.exp(-w2 * z * z)

    return pl.pallas_call(
        body,
        grid=(grid,),
        in_specs=[
            pl.BlockSpec((blk, 2), lambda i: (i, 0)),
            pl.BlockSpec((1, n_rbf), lambda i: (0, 0)),
            pl.BlockSpec((1, n_rbf), lambda i: (0, 0)),
        ],
        out_specs=[
            pl.BlockSpec((blk, 2), lambda i: (i, 0)),
            pl.BlockSpec((blk, 2), lambda i: (i, 0)),
            pl.BlockSpec((blk, 2 * n_rbf), lambda i: (i, 0)),
        ],
        out_shape=[
            jax.ShapeDtypeStruct((half, 2), jnp.float32),
            jax.ShapeDtypeStruct((half, 2), jnp.float32),
            jax.ShapeDtypeStruct((half, 2 * n_rbf), jnp.float32),
        ],
    )(d2_pairs, centers_row, widths_row)


def _tc_features(ids_pairs, table):
    half, _ = ids_pairs.shape
    v, d = table.shape
    blk = 1000
    grid = half // blk

    def body(id_ref, t_ref, out_ref):
        ids = id_ref[...]
        iot = lax.broadcasted_iota(jnp.int32, (1, v), 1)
        t = t_ref[...]
        fe = jnp.dot((ids[:, 0:1] == iot).astype(jnp.float32), t,
                     preferred_element_type=jnp.float32,
                     precision=lax.Precision.HIGHEST)
        fo = jnp.dot((ids[:, 1:2] == iot).astype(jnp.float32), t,
                     preferred_element_type=jnp.float32,
                     precision=lax.Precision.HIGHEST)
        out_ref[...] = jnp.concatenate([fe, fo], axis=1)

    return pl.pallas_call(
        body,
        grid=(grid,),
        in_specs=[
            pl.BlockSpec((blk, 2), lambda i: (i, 0)),
            pl.BlockSpec((v, d), lambda i: (0, 0)),
        ],
        out_specs=pl.BlockSpec((blk, 2 * d), lambda i: (i, 0)),
        out_shape=jax.ShapeDtypeStruct((half, 2 * d), jnp.float32),
    )(ids_pairs, table)


def kernel(atomic_numbers, positions, idx_i, idx_j,
           atom_features, rbf_centers, rbf_widths):
    p = idx_i.shape[0]
    n_rbf = rbf_centers.shape[0]

    step = NUM_WORKERS * LANES
    p_pad = -(-p // step) * step
    pad = p_pad - p
    ii = idx_i.astype(jnp.int32)
    jj = idx_j.astype(jnp.int32)
    if pad:
        zeros = jnp.zeros((pad,), jnp.int32)
        ii = jnp.concatenate([ii, zeros])
        jj = jnp.concatenate([jj, zeros])

    pos = positions.astype(jnp.float32)
    d2 = _sc_pair_dist2(pos[:, 0], pos[:, 1], pos[:, 2], ii, jj)[:p]

    d_half, cut_half, rbf_half = _tc_rbf(
        d2.reshape(p // 2, 2),
        rbf_centers.astype(jnp.float32).reshape(1, n_rbf),
        rbf_widths.astype(jnp.float32).reshape(1, n_rbf),
    )
    distances = d_half.reshape(p)
    cutoffs = cut_half.reshape(p)
    rbfs = rbf_half.reshape(p, n_rbf)

    vocab, dim = atom_features.shape
    v_pad = -(-vocab // 8) * 8
    table = atom_features.astype(jnp.float32)
    if v_pad != vocab:
        table = jnp.concatenate(
            [table, jnp.zeros((v_pad - vocab, dim), jnp.float32)])
    n = atomic_numbers.shape[0]
    feat_half = _tc_features(
        atomic_numbers.astype(jnp.int32).reshape(n // 2, 2), table)
    features = feat_half.reshape(n, dim)

    return (features, distances, cutoffs, rbfs, distances)


# speedup vs baseline: 6.8865x; 2.1851x over previous
import functools

import jax
import jax.numpy as jnp
from jax import lax
from jax.experimental import pallas as pl
from jax.experimental.pallas import tpu as pltpu
from jax.experimental.pallas import tpu_sc as plsc

CUTOFF = 8.0
NUM_WORKERS = 32
LANES = 16


def _sc_pair_dist2(xs, ys, zs, idx_i, idx_j):
    n_atoms = xs.shape[0]
    p = idx_i.shape[0]
    per_w = -(-p // NUM_WORKERS)
    per_w = -(-per_w // LANES) * LANES
    chunks = per_w // LANES
    mesh = plsc.VectorSubcoreMesh(core_axis_name="c", subcore_axis_name="s")

    @functools.partial(
        pl.kernel,
        out_type=jax.ShapeDtypeStruct((p,), jnp.float32),
        mesh=mesh,
        scratch_types=[
            pltpu.VMEM((n_atoms,), jnp.float32),
            pltpu.VMEM((per_w,), jnp.int32),
            pltpu.VMEM((per_w,), jnp.int32),
            pltpu.VMEM((per_w,), jnp.float32),
        ],
        compiler_params=pltpu.CompilerParams(needs_layout_passes=False),
    )
    def sc_kernel(x_hbm, y_hbm, z_hbm, ii_hbm, jj_hbm, d2_hbm,
                  tab_v, ii_v, jj_v, d2_v):
        wid = lax.axis_index("s") * 2 + lax.axis_index("c")
        base = jnp.minimum(wid * per_w, p - per_w)
        pltpu.sync_copy(ii_hbm.at[pl.ds(base, per_w)], ii_v)
        pltpu.sync_copy(jj_hbm.at[pl.ds(base, per_w)], jj_v)
        for c, coord_hbm in enumerate((x_hbm, y_hbm, z_hbm)):
            pltpu.sync_copy(coord_hbm, tab_v)

            def body(k, _, first=(c == 0)):
                off = k * LANES
                ii = ii_v[pl.ds(off, LANES)]
                jj = jj_v[pl.ds(off, LANES)]
                xi = plsc.load_gather(tab_v, [ii])
                xj = plsc.load_gather(tab_v, [jj])
                d = xj - xi
                if first:
                    d2_v[pl.ds(off, LANES)] = d * d
                else:
                    d2_v[pl.ds(off, LANES)] = d2_v[pl.ds(off, LANES)] + d * d
                return 0

            lax.fori_loop(0, chunks, body, 0)
        pltpu.sync_copy(d2_v, d2_hbm.at[pl.ds(base, per_w)])

    return sc_kernel(xs, ys, zs, idx_i, idx_j)


def _sc_features(ids, table):
    n = ids.shape[0]
    v, d = table.shape
    per_w = 1600
    n_chunks = 4
    chunk = per_w // n_chunks
    mesh = plsc.VectorSubcoreMesh(core_axis_name="c", subcore_axis_name="s")

    @functools.partial(
        pl.kernel,
        out_type=jax.ShapeDtypeStruct((n, d), jnp.float32),
        mesh=mesh,
        scratch_types=[
            pltpu.VMEM((per_w,), jnp.int32),
            pltpu.VMEM((chunk, d), jnp.float32),
            pltpu.SemaphoreType.DMA,
        ],
        compiler_params=pltpu.CompilerParams(needs_layout_passes=False),
    )
    def feat_kernel(ids_hbm, tab_hbm, out_hbm, ids_v, rows_v, sem):
        wid = lax.axis_index("s") * 2 + lax.axis_index("c")
        base = jnp.minimum(wid * per_w, n - per_w)
        pltpu.sync_copy(ids_hbm.at[pl.ds(base, per_w)], ids_v)
        for k in range(n_chunks):
            pltpu.async_copy(
                tab_hbm.at[ids_v.at[pl.ds(k * chunk, chunk)]],
                rows_v, sem).wait()
            pltpu.sync_copy(rows_v, out_hbm.at[pl.ds(base + k * chunk, chunk), :])

    return feat_kernel(ids, table)


def _tc_rbf(d2_sq, centers_row, widths_row, n_rbf):
    rows, w128 = d2_sq.shape
    r_blk = 32
    grid = -(-rows // r_blk)

    def body(d2_ref, c_ref, w_ref, d_ref, cut_ref, rbf_ref):
        d2 = d2_ref[...]
        dd = jnp.sqrt(d2)
        d_ref[...] = dd
        x = dd * (1.0 / CUTOFF)
        x3 = x * x * x
        f = 1.0 + x3 * (-10.0 + x * (15.0 - 6.0 * x))
        cut_ref[...] = jnp.where(dd < CUTOFF, f, jnp.zeros_like(f))
        c_col = c_ref[...]
        w_col = w_ref[...]
        pieces = []
        for r in range(r_blk):
            z = dd[r:r + 1, :] - c_col
            t = jnp.exp(-w_col * z * z)
            pieces.append(jnp.transpose(t))
        rbf_ref[...] = jnp.concatenate(pieces, axis=0)

    return pl.pallas_call(
        body,
        grid=(grid,),
        in_specs=[
            pl.BlockSpec((r_blk, w128), lambda i: (i, 0)),
            pl.BlockSpec((n_rbf, 1), lambda i: (0, 0)),
            pl.BlockSpec((n_rbf, 1), lambda i: (0, 0)),
        ],
        out_specs=[
            pl.BlockSpec((r_blk, w128), lambda i: (i, 0)),
            pl.BlockSpec((r_blk, w128), lambda i: (i, 0)),
            pl.BlockSpec((r_blk * w128, n_rbf), lambda i: (i, 0)),
        ],
        out_shape=[
            jax.ShapeDtypeStruct((rows, w128), jnp.float32),
            jax.ShapeDtypeStruct((rows, w128), jnp.float32),
            jax.ShapeDtypeStruct((rows * w128, n_rbf), jnp.float32),
        ],
    )(d2_sq, centers_row, widths_row)


def kernel(atomic_numbers, positions, idx_i, idx_j,
           atom_features, rbf_centers, rbf_widths):
    p = idx_i.shape[0]
    n_rbf = rbf_centers.shape[0]

    ii = idx_i.astype(jnp.int32)
    jj = idx_j.astype(jnp.int32)
    pos = positions.astype(jnp.float32)
    d2 = _sc_pair_dist2(pos[:, 0], pos[:, 1], pos[:, 2], ii, jj)

    d_sq, cut_sq, rbfs = _tc_rbf(
        d2.reshape(p // 128, 128),
        rbf_centers.astype(jnp.float32).reshape(n_rbf, 1),
        rbf_widths.astype(jnp.float32).reshape(n_rbf, 1),
        n_rbf,
    )
    distances = d_sq.reshape(p)
    cutoffs = cut_sq.reshape(p)

    features = _sc_features(atomic_numbers.astype(jnp.int32),
                            atom_features.astype(jnp.float32))

    return (features, distances, cutoffs, rbfs, distances)


# speedup vs baseline: 15.4820x; 2.2482x over previous
import functools

import jax
import jax.numpy as jnp
from jax import lax
from jax.experimental import pallas as pl
from jax.experimental.pallas import tpu as pltpu
from jax.experimental.pallas import tpu_sc as plsc

CUTOFF = 8.0
NUM_WORKERS = 32
LANES = 16


def _sc_pair_dist2(xs, ys, zs, idx_i, idx_j):
    n_atoms = xs.shape[0]
    p = idx_i.shape[0]
    per_w = -(-p // NUM_WORKERS)
    per_w = -(-per_w // LANES) * LANES
    chunks = per_w // LANES
    mesh = plsc.VectorSubcoreMesh(core_axis_name="c", subcore_axis_name="s")

    @functools.partial(
        pl.kernel,
        out_type=jax.ShapeDtypeStruct((p,), jnp.float32),
        mesh=mesh,
        scratch_types=[
            pltpu.VMEM((n_atoms,), jnp.float32),
            pltpu.VMEM((per_w,), jnp.int32),
            pltpu.VMEM((per_w,), jnp.int32),
            pltpu.VMEM((per_w,), jnp.float32),
        ],
        compiler_params=pltpu.CompilerParams(needs_layout_passes=False),
    )
    def sc_kernel(x_hbm, y_hbm, z_hbm, ii_hbm, jj_hbm, d2_hbm,
                  tab_v, ii_v, jj_v, d2_v):
        wid = lax.axis_index("s") * 2 + lax.axis_index("c")
        base = jnp.minimum(wid * per_w, p - per_w)
        pltpu.sync_copy(ii_hbm.at[pl.ds(base, per_w)], ii_v)
        pltpu.sync_copy(jj_hbm.at[pl.ds(base, per_w)], jj_v)
        for c, coord_hbm in enumerate((x_hbm, y_hbm, z_hbm)):
            pltpu.sync_copy(coord_hbm, tab_v)

            def body(k, _, first=(c == 0)):
                off = k * LANES
                ii = ii_v[pl.ds(off, LANES)]
                jj = jj_v[pl.ds(off, LANES)]
                xi = plsc.load_gather(tab_v, [ii])
                xj = plsc.load_gather(tab_v, [jj])
                d = xj - xi
                if first:
                    d2_v[pl.ds(off, LANES)] = d * d
                else:
                    d2_v[pl.ds(off, LANES)] = d2_v[pl.ds(off, LANES)] + d * d
                return 0

            lax.fori_loop(0, chunks, body, 0)
        pltpu.sync_copy(d2_v, d2_hbm.at[pl.ds(base, per_w)])

    return sc_kernel(xs, ys, zs, idx_i, idx_j)


def _sc_features(ids, table):
    n = ids.shape[0]
    v, d = table.shape
    per_w = 1600
    n_chunks = 4
    chunk = per_w // n_chunks
    mesh = plsc.VectorSubcoreMesh(core_axis_name="c", subcore_axis_name="s")

    @functools.partial(
        pl.kernel,
        out_type=jax.ShapeDtypeStruct((n, d), jnp.float32),
        mesh=mesh,
        scratch_types=[
            pltpu.VMEM((per_w,), jnp.int32),
            pltpu.VMEM((chunk, d), jnp.float32),
            pltpu.SemaphoreType.DMA,
        ],
        compiler_params=pltpu.CompilerParams(needs_layout_passes=False),
    )
    def feat_kernel(ids_hbm, tab_hbm, out_hbm, ids_v, rows_v, sem):
        wid = lax.axis_index("s") * 2 + lax.axis_index("c")
        base = jnp.minimum(wid * per_w, n - per_w)
        pltpu.sync_copy(ids_hbm.at[pl.ds(base, per_w)], ids_v)
        for k in range(n_chunks):
            pltpu.async_copy(
                tab_hbm.at[ids_v.at[pl.ds(k * chunk, chunk)]],
                rows_v, sem).wait()
            pltpu.sync_copy(rows_v, out_hbm.at[pl.ds(base + k * chunk, chunk), :])

    return feat_kernel(ids, table)


def _tc_rbf(d2_sq, centers_row, widths_row, n_rbf):
    rows, w128 = d2_sq.shape
    r_blk = 32
    grid = -(-rows // r_blk)

    def body(d2_ref, c_ref, w_ref, d_ref, cut_ref, rbf_ref):
        d2 = d2_ref[...]
        dd = jnp.sqrt(d2)
        d_ref[...] = dd
        x = dd * (1.0 / CUTOFF)
        x3 = x * x * x
        f = 1.0 + x3 * (-10.0 + x * (15.0 - 6.0 * x))
        cut_ref[...] = jnp.where(dd < CUTOFF, f, jnp.zeros_like(f))
        c_col = c_ref[...]
        w_col = w_ref[...]
        pieces = []
        for r in range(r_blk):
            z = dd[r:r + 1, :] - c_col
            pieces.append(jnp.exp(-w_col * z * z))
        rbf_ref[...] = jnp.concatenate(pieces, axis=1)

    return pl.pallas_call(
        body,
        grid=(grid,),
        in_specs=[
            pl.BlockSpec((r_blk, w128), lambda i: (i, 0)),
            pl.BlockSpec((n_rbf, 1), lambda i: (0, 0)),
            pl.BlockSpec((n_rbf, 1), lambda i: (0, 0)),
        ],
        out_specs=[
            pl.BlockSpec((r_blk, w128), lambda i: (i, 0)),
            pl.BlockSpec((r_blk, w128), lambda i: (i, 0)),
            pl.BlockSpec((n_rbf, r_blk * w128), lambda i: (0, i)),
        ],
        out_shape=[
            jax.ShapeDtypeStruct((rows, w128), jnp.float32),
            jax.ShapeDtypeStruct((rows, w128), jnp.float32),
            jax.ShapeDtypeStruct((n_rbf, rows * w128), jnp.float32),
        ],
    )(d2_sq, centers_row, widths_row)


def kernel(atomic_numbers, positions, idx_i, idx_j,
           atom_features, rbf_centers, rbf_widths):
    p = idx_i.shape[0]
    n_rbf = rbf_centers.shape[0]

    ii = idx_i.astype(jnp.int32)
    jj = idx_j.astype(jnp.int32)
    pos = positions.astype(jnp.float32)
    d2 = _sc_pair_dist2(pos[:, 0], pos[:, 1], pos[:, 2], ii, jj)

    d_sq, cut_sq, rbfs_t = _tc_rbf(
        d2.reshape(p // 128, 128),
        rbf_centers.astype(jnp.float32).reshape(n_rbf, 1),
        rbf_widths.astype(jnp.float32).reshape(n_rbf, 1),
        n_rbf,
    )
    distances = d_sq.reshape(p)
    cutoffs = cut_sq.reshape(p)
    rbfs = jnp.transpose(rbfs_t)

    features = _sc_features(atomic_numbers.astype(jnp.int32),
                            atom_features.astype(jnp.float32))

    return (features, distances, cutoffs, rbfs, distances)


# speedup vs baseline: 18.0874x; 1.1683x over previous
import functools

import jax
import jax.numpy as jnp
from jax import lax
from jax.experimental import pallas as pl
from jax.experimental.pallas import tpu as pltpu
from jax.experimental.pallas import tpu_sc as plsc

CUTOFF = 8.0
NUM_WORKERS = 32
LANES = 16


def _sc_pair_dist2(xs, ys, zs, idx_i, idx_j):
    n_atoms = xs.shape[0]
    p = idx_i.shape[0]
    per_w = -(-p // NUM_WORKERS)
    per_w = -(-per_w // LANES) * LANES
    chunks = per_w // LANES
    mesh = plsc.VectorSubcoreMesh(core_axis_name="c", subcore_axis_name="s")

    @functools.partial(
        pl.kernel,
        out_type=jax.ShapeDtypeStruct((p,), jnp.float32),
        mesh=mesh,
        scratch_types=[
            pltpu.VMEM((n_atoms,), jnp.float32),
            pltpu.VMEM((per_w,), jnp.int32),
            pltpu.VMEM((per_w,), jnp.int32),
            pltpu.VMEM((per_w,), jnp.float32),
        ],
        compiler_params=pltpu.CompilerParams(needs_layout_passes=False),
    )
    def sc_kernel(x_hbm, y_hbm, z_hbm, ii_hbm, jj_hbm, d2_hbm,
                  tab_v, ii_v, jj_v, d2_v):
        wid = lax.axis_index("s") * 2 + lax.axis_index("c")
        base = jnp.minimum(wid * per_w, p - per_w)
        pltpu.sync_copy(ii_hbm.at[pl.ds(base, per_w)], ii_v)
        pltpu.sync_copy(jj_hbm.at[pl.ds(base, per_w)], jj_v)
        for c, coord_hbm in enumerate((x_hbm, y_hbm, z_hbm)):
            pltpu.sync_copy(coord_hbm, tab_v)

            def body(k, _, first=(c == 0)):
                off = k * LANES
                ii = ii_v[pl.ds(off, LANES)]
                jj = jj_v[pl.ds(off, LANES)]
                xi = plsc.load_gather(tab_v, [ii])
                xj = plsc.load_gather(tab_v, [jj])
                d = xj - xi
                if first:
                    d2_v[pl.ds(off, LANES)] = d * d
                else:
                    d2_v[pl.ds(off, LANES)] = d2_v[pl.ds(off, LANES)] + d * d
                return 0

            lax.fori_loop(0, chunks, body, 0)
        pltpu.sync_copy(d2_v, d2_hbm.at[pl.ds(base, per_w)])

    return sc_kernel(xs, ys, zs, idx_i, idx_j)


def _sc_features(ids, table):
    n = ids.shape[0]
    v, d = table.shape
    per_w = 1600
    n_chunks = 4
    chunk = per_w // n_chunks
    mesh = plsc.VectorSubcoreMesh(core_axis_name="c", subcore_axis_name="s")

    @functools.partial(
        pl.kernel,
        out_type=jax.ShapeDtypeStruct((n, d), jnp.float32),
        mesh=mesh,
        scratch_types=[
            pltpu.VMEM((per_w,), jnp.int32),
            pltpu.VMEM((chunk, d), jnp.float32),
            pltpu.SemaphoreType.DMA,
        ],
        compiler_params=pltpu.CompilerParams(needs_layout_passes=False),
    )
    def feat_kernel(ids_hbm, tab_hbm, out_hbm, ids_v, rows_v, sem):
        wid = lax.axis_index("s") * 2 + lax.axis_index("c")
        base = jnp.minimum(wid * per_w, n - per_w)
        pltpu.sync_copy(ids_hbm.at[pl.ds(base, per_w)], ids_v)
        for k in range(n_chunks):
            pltpu.async_copy(
                tab_hbm.at[ids_v.at[pl.ds(k * chunk, chunk)]],
                rows_v, sem).wait()
            pltpu.sync_copy(rows_v, out_hbm.at[pl.ds(base + k * chunk, chunk), :])

    return feat_kernel(ids, table)


def _tc_rbf(d2_sq, centers_row, widths_row, n_rbf):
    rows, w128 = d2_sq.shape
    r_blk = 64
    grid = -(-rows // r_blk)

    def body(d2_ref, c_ref, w_ref, d_ref, cut_ref, rbf_ref):
        d2 = d2_ref[...]
        dd = jnp.sqrt(d2)
        d_ref[...] = dd
        x = dd * (1.0 / CUTOFF)
        x3 = x * x * x
        f = 1.0 + x3 * (-10.0 + x * (15.0 - 6.0 * x))
        cut_ref[...] = jnp.where(dd < CUTOFF, f, jnp.zeros_like(f))
        c_col = c_ref[...]
        w_col = w_ref[...]
        pieces = []
        for r in range(r_blk):
            z = dd[r:r + 1, :] - c_col
            pieces.append(jnp.exp(-w_col * z * z))
        rbf_ref[...] = jnp.concatenate(pieces, axis=1)

    return pl.pallas_call(
        body,
        grid=(grid,),
        in_specs=[
            pl.BlockSpec((r_blk, w128), lambda i: (i, 0)),
            pl.BlockSpec((n_rbf, 1), lambda i: (0, 0)),
            pl.BlockSpec((n_rbf, 1), lambda i: (0, 0)),
        ],
        out_specs=[
            pl.BlockSpec((r_blk, w128), lambda i: (i, 0)),
            pl.BlockSpec((r_blk, w128), lambda i: (i, 0)),
            pl.BlockSpec((n_rbf, r_blk * w128), lambda i: (0, i)),
        ],
        out_shape=[
            jax.ShapeDtypeStruct((rows, w128), jnp.float32),
            jax.ShapeDtypeStruct((rows, w128), jnp.float32),
            jax.ShapeDtypeStruct((n_rbf, rows * w128), jnp.float32),
        ],
    )(d2_sq, centers_row, widths_row)


def kernel(atomic_numbers, positions, idx_i, idx_j,
           atom_features, rbf_centers, rbf_widths):
    p = idx_i.shape[0]
    n_rbf = rbf_centers.shape[0]

    ii = idx_i.astype(jnp.int32)
    jj = idx_j.astype(jnp.int32)
    pos = positions.astype(jnp.float32)
    d2 = _sc_pair_dist2(pos[:, 0], pos[:, 1], pos[:, 2], ii, jj)

    d_sq, cut_sq, rbfs_t = _tc_rbf(
        d2.reshape(p // 128, 128),
        rbf_centers.astype(jnp.float32).reshape(n_rbf, 1),
        rbf_widths.astype(jnp.float32).reshape(n_rbf, 1),
        n_rbf,
    )
    distances = d_sq.reshape(p)
    cutoffs = cut_sq.reshape(p)
    rbfs = jnp.transpose(rbfs_t)

    features = _sc_features(atomic_numbers.astype(jnp.int32),
                            atom_features.astype(jnp.float32))

    return (features, distances, cutoffs, rbfs, distances)


# speedup vs baseline: 18.8391x; 1.0416x over previous
import functools

import jax
import jax.numpy as jnp
from jax import lax
from jax.experimental import pallas as pl
from jax.experimental.pallas import tpu as pltpu
from jax.experimental.pallas import tpu_sc as plsc

CUTOFF = 8.0
NUM_WORKERS = 32
LANES = 16


def _sc_pair_dist2(xs, ys, zs, idx_i, idx_j):
    n_atoms = xs.shape[0]
    p = idx_i.shape[0]
    per_w = -(-p // NUM_WORKERS)
    per_w = -(-per_w // LANES) * LANES
    chunks = per_w // LANES
    mesh = plsc.VectorSubcoreMesh(core_axis_name="c", subcore_axis_name="s")

    @functools.partial(
        pl.kernel,
        out_type=jax.ShapeDtypeStruct((p,), jnp.float32),
        mesh=mesh,
        scratch_types=[
            pltpu.VMEM((n_atoms,), jnp.float32),
            pltpu.VMEM((per_w,), jnp.int32),
            pltpu.VMEM((per_w,), jnp.int32),
            pltpu.VMEM((per_w,), jnp.float32),
        ],
        compiler_params=pltpu.CompilerParams(needs_layout_passes=False),
    )
    def sc_kernel(x_hbm, y_hbm, z_hbm, ii_hbm, jj_hbm, d2_hbm,
                  tab_v, ii_v, jj_v, d2_v):
        wid = lax.axis_index("s") * 2 + lax.axis_index("c")
        base = jnp.minimum(wid * per_w, p - per_w)
        pltpu.sync_copy(ii_hbm.at[pl.ds(base, per_w)], ii_v)
        pltpu.sync_copy(jj_hbm.at[pl.ds(base, per_w)], jj_v)
        for c, coord_hbm in enumerate((x_hbm, y_hbm, z_hbm)):
            pltpu.sync_copy(coord_hbm, tab_v)

            def body(k, _, first=(c == 0)):
                off = k * LANES
                ii = ii_v[pl.ds(off, LANES)]
                jj = jj_v[pl.ds(off, LANES)]
                xi = plsc.load_gather(tab_v, [ii])
                xj = plsc.load_gather(tab_v, [jj])
                d = xj - xi
                if first:
                    d2_v[pl.ds(off, LANES)] = d * d
                else:
                    d2_v[pl.ds(off, LANES)] = d2_v[pl.ds(off, LANES)] + d * d
                return 0

            lax.fori_loop(0, chunks, body, 0)
        pltpu.sync_copy(d2_v, d2_hbm.at[pl.ds(base, per_w)])

    return sc_kernel(xs, ys, zs, idx_i, idx_j)


def _sc_features(ids, table):
    n = ids.shape[0]
    v, d = table.shape
    per_w = 1600
    n_chunks = 4
    chunk = per_w // n_chunks
    mesh = plsc.VectorSubcoreMesh(core_axis_name="c", subcore_axis_name="s")

    @functools.partial(
        pl.kernel,
        out_type=jax.ShapeDtypeStruct((n, d), jnp.float32),
        mesh=mesh,
        scratch_types=[
            pltpu.VMEM((per_w,), jnp.int32),
            pltpu.VMEM((chunk, d), jnp.float32),
            pltpu.SemaphoreType.DMA,
        ],
        compiler_params=pltpu.CompilerParams(needs_layout_passes=False),
    )
    def feat_kernel(ids_hbm, tab_hbm, out_hbm, ids_v, rows_v, sem):
        wid = lax.axis_index("s") * 2 + lax.axis_index("c")
        base = jnp.minimum(wid * per_w, n - per_w)
        pltpu.sync_copy(ids_hbm.at[pl.ds(base, per_w)], ids_v)
        for k in range(n_chunks):
            pltpu.async_copy(
                tab_hbm.at[ids_v.at[pl.ds(k * chunk, chunk)]],
                rows_v, sem).wait()
            pltpu.sync_copy(rows_v, out_hbm.at[pl.ds(base + k * chunk, chunk), :])

    return feat_kernel(ids, table)


def _tc_rbf(d2_sq, centers_row, widths_row, n_rbf):
    rows, w128 = d2_sq.shape
    r_blk = 128
    grid = -(-rows // r_blk)

    def body(d2_ref, c_ref, w_ref, d_ref, cut_ref, rbf_ref):
        d2 = d2_ref[...]
        dd = jnp.sqrt(d2)
        d_ref[...] = dd
        x = dd * (1.0 / CUTOFF)
        x3 = x * x * x
        f = 1.0 + x3 * (-10.0 + x * (15.0 - 6.0 * x))
        cut_ref[...] = jnp.where(dd < CUTOFF, f, jnp.zeros_like(f))
        c_col = c_ref[...]
        w_col = w_ref[...]
        pieces = []
        for r in range(r_blk):
            z = dd[r:r + 1, :] - c_col
            pieces.append(jnp.exp(-w_col * z * z))
        rbf_ref[...] = jnp.concatenate(pieces, axis=1)

    return pl.pallas_call(
        body,
        grid=(grid,),
        in_specs=[
            pl.BlockSpec((r_blk, w128), lambda i: (i, 0)),
            pl.BlockSpec((n_rbf, 1), lambda i: (0, 0)),
            pl.BlockSpec((n_rbf, 1), lambda i: (0, 0)),
        ],
        out_specs=[
            pl.BlockSpec((r_blk, w128), lambda i: (i, 0)),
            pl.BlockSpec((r_blk, w128), lambda i: (i, 0)),
            pl.BlockSpec((n_rbf, r_blk * w128), lambda i: (0, i)),
        ],
        out_shape=[
            jax.ShapeDtypeStruct((rows, w128), jnp.float32),
            jax.ShapeDtypeStruct((rows, w128), jnp.float32),
            jax.ShapeDtypeStruct((n_rbf, rows * w128), jnp.float32),
        ],
    )(d2_sq, centers_row, widths_row)


def kernel(atomic_numbers, positions, idx_i, idx_j,
           atom_features, rbf_centers, rbf_widths):
    p = idx_i.shape[0]
    n_rbf = rbf_centers.shape[0]

    ii = idx_i.astype(jnp.int32)
    jj = idx_j.astype(jnp.int32)
    pos = positions.astype(jnp.float32)
    d2 = _sc_pair_dist2(pos[:, 0], pos[:, 1], pos[:, 2], ii, jj)

    d_sq, cut_sq, rbfs_t = _tc_rbf(
        d2.reshape(p // 128, 128),
        rbf_centers.astype(jnp.float32).reshape(n_rbf, 1),
        rbf_widths.astype(jnp.float32).reshape(n_rbf, 1),
        n_rbf,
    )
    distances = d_sq.reshape(p)
    cutoffs = cut_sq.reshape(p)
    rbfs = jnp.transpose(rbfs_t)

    features = _sc_features(atomic_numbers.astype(jnp.int32),
                            atom_features.astype(jnp.float32))

    return (features, distances, cutoffs, rbfs, distances)


# speedup vs baseline: 21.1775x; 1.1241x over previous
import functools

import jax
import jax.numpy as jnp
from jax import lax
from jax.experimental import pallas as pl
from jax.experimental.pallas import tpu as pltpu
from jax.experimental.pallas import tpu_sc as plsc

CUTOFF = 8.0
NUM_WORKERS = 32
LANES = 16


def _sc_pair_dist2(xs, ys, zs, idx_i, idx_j):
    n_atoms = xs.shape[0]
    p = idx_i.shape[0]
    per_w = -(-p // NUM_WORKERS)
    per_w = -(-per_w // LANES) * LANES
    chunks = per_w // LANES
    mesh = plsc.VectorSubcoreMesh(core_axis_name="c", subcore_axis_name="s")

    @functools.partial(
        pl.kernel,
        out_type=jax.ShapeDtypeStruct((p,), jnp.float32),
        mesh=mesh,
        scratch_types=[
            pltpu.VMEM((n_atoms,), jnp.float32),
            pltpu.VMEM((per_w,), jnp.int32),
            pltpu.VMEM((per_w,), jnp.int32),
            pltpu.VMEM((per_w,), jnp.float32),
        ],
        compiler_params=pltpu.CompilerParams(needs_layout_passes=False),
    )
    def sc_kernel(x_hbm, y_hbm, z_hbm, ii_hbm, jj_hbm, d2_hbm,
                  tab_v, ii_v, jj_v, d2_v):
        wid = lax.axis_index("s") * 2 + lax.axis_index("c")
        base = jnp.minimum(wid * per_w, p - per_w)
        pltpu.sync_copy(ii_hbm.at[pl.ds(base, per_w)], ii_v)
        pltpu.sync_copy(jj_hbm.at[pl.ds(base, per_w)], jj_v)
        for c, coord_hbm in enumerate((x_hbm, y_hbm, z_hbm)):
            pltpu.sync_copy(coord_hbm, tab_v)
            first = c == 0

            @plsc.parallel_loop(0, per_w, step=LANES, unroll=8)
            def _(off, _first=first):
                ii = ii_v[pl.ds(off, LANES)]
                jj = jj_v[pl.ds(off, LANES)]
                xi = plsc.load_gather(tab_v, [ii])
                xj = plsc.load_gather(tab_v, [jj])
                d = xj - xi
                if _first:
                    d2_v[pl.ds(off, LANES)] = d * d
                else:
                    d2_v[pl.ds(off, LANES)] = d2_v[pl.ds(off, LANES)] + d * d
        pltpu.sync_copy(d2_v, d2_hbm.at[pl.ds(base, per_w)])

    return sc_kernel(xs, ys, zs, idx_i, idx_j)


def _sc_features(ids, table):
    n = ids.shape[0]
    v, d = table.shape
    per_w = 1600
    n_chunks = 4
    chunk = per_w // n_chunks
    mesh = plsc.VectorSubcoreMesh(core_axis_name="c", subcore_axis_name="s")

    @functools.partial(
        pl.kernel,
        out_type=jax.ShapeDtypeStruct((n, d), jnp.float32),
        mesh=mesh,
        scratch_types=[
            pltpu.VMEM((per_w,), jnp.int32),
            pltpu.VMEM((chunk, d), jnp.float32),
            pltpu.SemaphoreType.DMA,
        ],
        compiler_params=pltpu.CompilerParams(needs_layout_passes=False),
    )
    def feat_kernel(ids_hbm, tab_hbm, out_hbm, ids_v, rows_v, sem):
        wid = lax.axis_index("s") * 2 + lax.axis_index("c")
        base = jnp.minimum(wid * per_w, n - per_w)
        pltpu.sync_copy(ids_hbm.at[pl.ds(base, per_w)], ids_v)
        for k in range(n_chunks):
            pltpu.async_copy(
                tab_hbm.at[ids_v.at[pl.ds(k * chunk, chunk)]],
                rows_v, sem).wait()
            pltpu.sync_copy(rows_v, out_hbm.at[pl.ds(base + k * chunk, chunk), :])

    return feat_kernel(ids, table)


def _tc_rbf(d2_sq, centers_row, widths_row, n_rbf):
    rows, w128 = d2_sq.shape
    r_blk = 128
    grid = -(-rows // r_blk)

    def body(d2_ref, c_ref, w_ref, d_ref, cut_ref, rbf_ref):
        d2 = d2_ref[...]
        dd = jnp.sqrt(d2)
        d_ref[...] = dd
        x = dd * (1.0 / CUTOFF)
        x3 = x * x * x
        f = 1.0 + x3 * (-10.0 + x * (15.0 - 6.0 * x))
        cut_ref[...] = jnp.where(dd < CUTOFF, f, jnp.zeros_like(f))
        c_col = c_ref[...]
        w_col = w_ref[...]
        pieces = []
        for r in range(r_blk):
            z = dd[r:r + 1, :] - c_col
            pieces.append(jnp.exp(-w_col * z * z))
        rbf_ref[...] = jnp.concatenate(pieces, axis=1)

    return pl.pallas_call(
        body,
        grid=(grid,),
        in_specs=[
            pl.BlockSpec((r_blk, w128), lambda i: (i, 0)),
            pl.BlockSpec((n_rbf, 1), lambda i: (0, 0)),
            pl.BlockSpec((n_rbf, 1), lambda i: (0, 0)),
        ],
        out_specs=[
            pl.BlockSpec((r_blk, w128), lambda i: (i, 0)),
            pl.BlockSpec((r_blk, w128), lambda i: (i, 0)),
            pl.BlockSpec((n_rbf, r_blk * w128), lambda i: (0, i)),
        ],
        out_shape=[
            jax.ShapeDtypeStruct((rows, w128), jnp.float32),
            jax.ShapeDtypeStruct((rows, w128), jnp.float32),
            jax.ShapeDtypeStruct((n_rbf, rows * w128), jnp.float32),
        ],
    )(d2_sq, centers_row, widths_row)


def kernel(atomic_numbers, positions, idx_i, idx_j,
           atom_features, rbf_centers, rbf_widths):
    p = idx_i.shape[0]
    n_rbf = rbf_centers.shape[0]

    ii = idx_i.astype(jnp.int32)
    jj = idx_j.astype(jnp.int32)
    pos = positions.astype(jnp.float32)
    d2 = _sc_pair_dist2(pos[:, 0], pos[:, 1], pos[:, 2], ii, jj)

    d_sq, cut_sq, rbfs_t = _tc_rbf(
        d2.reshape(p // 128, 128),
        rbf_centers.astype(jnp.float32).reshape(n_rbf, 1),
        rbf_widths.astype(jnp.float32).reshape(n_rbf, 1),
        n_rbf,
    )
    distances = d_sq.reshape(p)
    cutoffs = cut_sq.reshape(p)
    rbfs = jnp.transpose(rbfs_t)

    features = _sc_features(atomic_numbers.astype(jnp.int32),
                            atom_features.astype(jnp.float32))

    return (features, distances, cutoffs, rbfs, distances)
